# baseline (speedup 1.0000x reference)
import functools

import jax
import jax.numpy as jnp
from jax import lax
from jax.experimental import pallas as pl
from jax.experimental.pallas import tpu as pltpu
from jax.experimental.pallas import tpu_sc as plsc

BATCH = 4096
FIELDS = 26
NC = 2
NS = 16
NW = NC * NS
BPW = BATCH // NW
LANES = 16

_mesh = plsc.VectorSubcoreMesh(core_axis_name="c", subcore_axis_name="s")


@functools.partial(
    pl.kernel,
    mesh=_mesh,
    out_type=jax.ShapeDtypeStruct((BATCH,), jnp.float32),
    scratch_types=[
        pltpu.VMEM((FIELDS * BPW,), jnp.int32),
        pltpu.VMEM((FIELDS * BPW,), jnp.float32),
        pltpu.VMEM((BPW,), jnp.float32),
        pltpu.VMEM((LANES,), jnp.float32),
        pltpu.SemaphoreType.DMA,
    ],
)
def _sc_kernel(table_hbm, xr_hbm, bias_hbm, out_hbm,
               idxt_v, vals_v, out_v, bias_s, sem):
    wid = lax.axis_index("s") * NC + lax.axis_index("c")
    base = wid * BPW

    pltpu.sync_copy(bias_hbm, bias_s.at[pl.ds(0, 1)])
    idx_handles = [
        pltpu.async_copy(xr_hbm.at[j, pl.ds(base, BPW)],
                         idxt_v.at[pl.ds(j * BPW, BPW)], sem)
        for j in range(FIELDS)
    ]
    for h in idx_handles:
        h.wait()

    pltpu.async_copy(table_hbm.at[0].at[idxt_v], vals_v, sem).wait()

    bvec = lax.broadcast(bias_s[...][0], (LANES,))
    accs = [bvec for _ in range(BPW // LANES)]
    for j in range(FIELDS):
        for k in range(BPW // LANES):
            accs[k] = accs[k] + vals_v[pl.ds(j * BPW + k * LANES, LANES)]
    for k in range(BPW // LANES):
        out_v[pl.ds(k * LANES, LANES)] = accs[k]

    pltpu.sync_copy(out_v, out_hbm.at[pl.ds(base, BPW)])


def kernel(x, fc_weight, bias):
    table = fc_weight.T
    xr = x.T
    out = _sc_kernel(table, xr, bias)
    return out.reshape(BATCH, 1)


# speedup vs baseline: 1.0427x; 1.0427x over previous
import functools

import jax
import jax.numpy as jnp
from jax import lax
from jax.experimental import pallas as pl
from jax.experimental.pallas import tpu as pltpu
from jax.experimental.pallas import tpu_sc as plsc

BATCH = 4096
FIELDS = 26
VOCAB = 100000
NC = 2
NS = 16
NW = NC * NS
BPW = BATCH // NW
LANES = 16
HBM_FIELDS = 13
STAGE = 6144
TAIL_LO = (VOCAB // 128) * 128
TAIL_BULK = TAIL_LO - NS * STAGE
TAIL_N = VOCAB - TAIL_LO

_mesh = plsc.VectorSubcoreMesh(core_axis_name="c", subcore_axis_name="s")


@functools.partial(
    pl.kernel,
    mesh=_mesh,
    out_type=jax.ShapeDtypeStruct((BATCH,), jnp.float32),
    scratch_types=[
        pltpu.VMEM((FIELDS * BPW,), jnp.int32),
        pltpu.VMEM((FIELDS * BPW,), jnp.float32),
        pltpu.VMEM((BPW,), jnp.float32),
        pltpu.VMEM((LANES,), jnp.float32),
        pltpu.VMEM_SHARED((VOCAB,), jnp.float32),
        pltpu.VMEM((2 * LANES,), jnp.int32),
        pltpu.VMEM((2 * LANES,), jnp.float32),
        pltpu.SemaphoreType.DMA,
        pltpu.SemaphoreType.DMA,
        pltpu.SemaphoreType.DMA,
    ],
)
def _sc_kernel(table_hbm, xr_hbm, bias_hbm, out_hbm,
               idxt_v, vals_v, out_v, bias_s, table_sh,
               tidx_v, tval_v, sem, sem_stage, sem_sh):
    cid = lax.axis_index("c")
    sid = lax.axis_index("s")
    wid = sid * NC + cid
    base = wid * BPW

    lo = sid * STAGE
    stage_h = pltpu.async_copy(
        table_hbm.at[0, pl.ds(lo, STAGE)],
        table_sh.at[pl.ds(lo, STAGE)], sem_stage)

    @pl.when(sid == NS - 1)
    def _stage_bulk_tail():
        pltpu.sync_copy(
            table_hbm.at[0, pl.ds(NS * STAGE, TAIL_BULK)],
            table_sh.at[pl.ds(NS * STAGE, TAIL_BULK)])

    @pl.when(sid == 0)
    def _stage_last_words():
        it = lax.iota(jnp.int32, LANES)
        tidx_v[pl.ds(0, LANES)] = it + TAIL_LO
        tidx_v[pl.ds(LANES, LANES)] = it + (TAIL_LO + LANES)
        pltpu.async_copy(
            table_hbm.at[0].at[tidx_v.at[pl.ds(0, TAIL_N)]],
            tval_v.at[pl.ds(0, TAIL_N)], sem_stage).wait()
        pltpu.sync_copy(tval_v.at[pl.ds(0, TAIL_N)],
                        table_sh.at[pl.ds(TAIL_LO, TAIL_N)])

    pltpu.sync_copy(bias_hbm, bias_s.at[pl.ds(0, 1)])
    idx_handles = [
        pltpu.async_copy(xr_hbm.at[j, pl.ds(base, BPW)],
                         idxt_v.at[pl.ds(j * BPW, BPW)], sem)
        for j in range(FIELDS)
    ]
    for h in idx_handles:
        h.wait()

    nh = HBM_FIELDS * BPW
    hbm_h = pltpu.async_copy(
        table_hbm.at[0].at[idxt_v.at[pl.ds(0, nh)]],
        vals_v.at[pl.ds(0, nh)], sem)

    stage_h.wait()
    plsc.subcore_barrier()
    ns = (FIELDS - HBM_FIELDS) * BPW
    sh_h = pltpu.async_copy(
        table_sh.at[idxt_v.at[pl.ds(nh, ns)]],
        vals_v.at[pl.ds(nh, ns)], sem_sh)

    hbm_h.wait()
    sh_h.wait()

    bvec = lax.broadcast(bias_s[...][0], (LANES,))
    accs = [bvec for _ in range(BPW // LANES)]
    for j in range(FIELDS):
        for k in range(BPW // LANES):
            accs[k] = accs[k] + vals_v[pl.ds(j * BPW + k * LANES, LANES)]
    for k in range(BPW // LANES):
        out_v[pl.ds(k * LANES, LANES)] = accs[k]

    pltpu.sync_copy(out_v, out_hbm.at[pl.ds(base, BPW)])


def kernel(x, fc_weight, bias):
    table = fc_weight.T
    xr = x.T
    out = _sc_kernel(table, xr, bias)
    return out.reshape(BATCH, 1)
